# baseline (speedup 1.0000x reference)
import functools

import jax
import jax.numpy as jnp
from jax import lax
from jax.experimental import pallas as pl
from jax.experimental.pallas import tpu as pltpu
from jax.experimental.pallas import tpu_sc as plsc

BATCH = 16384
CF_DIM = 64
CD = 256
ALPHA = 0.5

NC = 2
NS = 16
NW = NC * NS
BPW = BATCH // NW

CF_SUB = 128
CF_NSUB = BPW // CF_SUB
PR_SUB = 64
PR_NSUB = BPW // PR_SUB

N_ROWS = 100000
N_PAD = 100096
H_ROWS = N_PAD // 2
PREP_R = 2176


def _mesh():
    return plsc.VectorSubcoreMesh(core_axis_name="c", subcore_axis_name="s",
                                  num_cores=NC, num_subcores=NS)


@functools.cache
def _make_sc_cfgather(n):
    bpw = n // NW
    nsub = bpw // CF_SUB

    @functools.partial(
        pl.kernel,
        out_type=[
            jax.ShapeDtypeStruct((n, 2 * CF_DIM), jnp.int32),
            jax.ShapeDtypeStruct((n, 2 * CF_DIM), jnp.int32),
        ],
        mesh=_mesh(),
        scratch_types=[
            pltpu.VMEM((bpw,), jnp.int32),
            pltpu.VMEM((bpw,), jnp.int32),
            pltpu.VMEM((bpw,), jnp.int32),
            pltpu.VMEM((bpw,), jnp.int32),
            pltpu.VMEM((2, CF_SUB, 2 * CF_DIM), jnp.int32),
            pltpu.VMEM((2, CF_SUB, 2 * CF_DIM), jnp.int32),
            pltpu.SemaphoreType.DMA,
            pltpu.SemaphoreType.DMA,
            pltpu.SemaphoreType.DMA,
            pltpu.SemaphoreType.DMA,
        ],
    )
    def _sc_cfgather(uids, iids, cfcat, dep, ucf_out, icf_out,
                     uid_v, iid_v, uhalf_v, ihalf_v, ubuf, ibuf,
                     su0, su1, si0, si1):
        del dep
        wid = lax.axis_index("s") * NC + lax.axis_index("c")
        base = wid * bpw
        pltpu.sync_copy(uids.at[pl.ds(base, bpw)], uid_v)
        pltpu.sync_copy(iids.at[pl.ds(base, bpw)], iid_v)

        def shift_body(i, _):
            uv = uid_v[pl.ds(i * 16, 16)]
            iv = iid_v[pl.ds(i * 16, 16)]
            uhalf_v[pl.ds(i * 16, 16)] = jnp.where(uv >= H_ROWS,
                                                   uv - H_ROWS, uv)
            ihalf_v[pl.ds(i * 16, 16)] = jnp.where(iv >= H_ROWS,
                                                   iv - H_ROWS, iv)
            return 0

        lax.fori_loop(0, bpw // 16, shift_body, 0)
        sem_u = (su0, su1)
        sem_i = (si0, si1)

        def fire(c):
            o = c * CF_SUB
            s = c % 2
            cu = pltpu.async_copy(cfcat.at[uhalf_v.at[pl.ds(o, CF_SUB)]],
                                  ubuf.at[s], sem_u[s])
            ci = pltpu.async_copy(cfcat.at[ihalf_v.at[pl.ds(o, CF_SUB)]],
                                  ibuf.at[s], sem_i[s])
            return cu, ci

        pend = fire(0)
        for c in range(nsub):
            cu, ci = pend
            if c + 1 < nsub:
                nxt = fire(c + 1)
            cu.wait()
            ci.wait()
            o = c * CF_SUB
            s = c % 2
            pltpu.sync_copy(ubuf.at[s], ucf_out.at[pl.ds(base + o, CF_SUB)])
            pltpu.sync_copy(ibuf.at[s], icf_out.at[pl.ds(base + o, CF_SUB)])
            if c + 1 < nsub:
                pend = nxt

    return _sc_cfgather


@functools.cache
def _make_sc_gather():
    @functools.partial(
        pl.kernel,
        out_type=[
            jax.ShapeDtypeStruct((BATCH, CD), jnp.float32),
            jax.ShapeDtypeStruct((BATCH, CD), jnp.float32),
        ],
        mesh=_mesh(),
        scratch_types=[
            pltpu.VMEM((BPW,), jnp.int32),
            pltpu.VMEM((BPW,), jnp.int32),
            pltpu.VMEM((2, PR_SUB, CD), jnp.float32),
            pltpu.VMEM((2, PR_SUB, CD), jnp.float32),
            pltpu.SemaphoreType.DMA,
            pltpu.SemaphoreType.DMA,
            pltpu.SemaphoreType.DMA,
            pltpu.SemaphoreType.DMA,
        ],
    )
    def _sc_gather(uids, iids, uprof, icont,
                   uprof_out, icont_out,
                   uid_v, iid_v, ubuf, ibuf, su0, su1, si0, si1):
        wid = lax.axis_index("s") * NC + lax.axis_index("c")
        base = wid * BPW
        pltpu.sync_copy(uids.at[pl.ds(base, BPW)], uid_v)
        pltpu.sync_copy(iids.at[pl.ds(base, BPW)], iid_v)
        sem_u = (su0, su1)
        sem_i = (si0, si1)

        def fire(c):
            o = c * PR_SUB
            s = c % 2
            cu = pltpu.async_copy(uprof.at[uid_v.at[pl.ds(o, PR_SUB)]],
                                  ubuf.at[s], sem_u[s])
            ci = pltpu.async_copy(icont.at[iid_v.at[pl.ds(o, PR_SUB)]],
                                  ibuf.at[s], sem_i[s])
            return cu, ci

        pend = fire(0)
        for c in range(PR_NSUB):
            cu, ci = pend
            if c + 1 < PR_NSUB:
                nxt = fire(c + 1)
            cu.wait()
            ci.wait()
            o = c * PR_SUB
            s = c % 2
            pltpu.sync_copy(ubuf.at[s], uprof_out.at[pl.ds(base + o, PR_SUB)])
            pltpu.sync_copy(ibuf.at[s], icont_out.at[pl.ds(base + o, PR_SUB)])
            if c + 1 < PR_NSUB:
                pend = nxt

    return _sc_gather


def _pack_pairs(x):
    xi = lax.bitcast_convert_type(x, jnp.int32)
    return ((xi[:, :32] >> 16) & jnp.int32(0xFFFF)) | (xi[:, 32:]
                                                      & jnp.int32(-65536))


def _tc_prep_body(ut1_ref, ut2_ref, it1_ref, it2_ref, out_ref):
    u1 = ut1_ref[...].T
    u2 = ut2_ref[...].T
    v1 = it1_ref[...].T
    v2 = it2_ref[...].T
    out_ref[...] = jnp.concatenate(
        [_pack_pairs(u1), _pack_pairs(v1), _pack_pairs(u2), _pack_pairs(v2)],
        axis=1)


_NB = H_ROWS // PREP_R


_tc_prep = pl.pallas_call(
    _tc_prep_body,
    grid=(_NB,),
    in_specs=[
        pl.BlockSpec((CF_DIM, PREP_R), lambda i: (0, i)),
        pl.BlockSpec((CF_DIM, PREP_R), lambda i: (0, i + _NB)),
        pl.BlockSpec((CF_DIM, PREP_R), lambda i: (0, i)),
        pl.BlockSpec((CF_DIM, PREP_R), lambda i: (0, i + _NB)),
    ],
    out_specs=pl.BlockSpec((PREP_R, 2 * CF_DIM), lambda i: (i, 0)),
    out_shape=jax.ShapeDtypeStruct((H_ROWS, 2 * CF_DIM), jnp.int32),
)


BLK = 1024


def _dot_packed(u32, i32):
    ue = lax.bitcast_convert_type(u32 << 16, jnp.float32)
    ie = lax.bitcast_convert_type(i32 << 16, jnp.float32)
    uo = lax.bitcast_convert_type(u32 & jnp.int32(-65536), jnp.float32)
    io = lax.bitcast_convert_type(i32 & jnp.int32(-65536), jnp.float32)
    return jnp.sum(ue * ie + uo * io, axis=1)


def _tc_body(uids_ref, iids_ref, ucf_ref, icf_ref, uprof_ref, icont_ref,
             w_ref, b_ref, g_ref, beta_ref, out_ref):
    u = uprof_ref[...]
    h = jnp.dot(u, w_ref[...], preferred_element_type=jnp.float32)
    h = h + b_ref[...]
    mu = jnp.mean(h, axis=1, keepdims=True)
    var = jnp.mean((h - mu) * (h - mu), axis=1, keepdims=True)
    hn = (h - mu) * lax.rsqrt(var + 1e-5) * g_ref[...] + beta_ref[...]
    hg = hn * 0.5 * (1.0 + lax.erf(hn * 0.7071067811865476))
    content = jnp.sum(hg * icont_ref[...], axis=1)
    uodd = uids_ref[...][:, None] >= H_ROWS
    iodd = iids_ref[...][:, None] >= H_ROWS
    ug = ucf_ref[...]
    ig = icf_ref[...]
    cf = _dot_packed(jnp.where(uodd, ug[:, 64:96], ug[:, 0:32]),
                     jnp.where(iodd, ig[:, 96:128], ig[:, 32:64]))
    out_ref[...] = ALPHA * cf + (1.0 - ALPHA) * content


@functools.cache
def _make_tc_score(n, row_off):
    ob = row_off // BLK
    return pl.pallas_call(
        _tc_body,
        grid=(n // BLK,),
        in_specs=[
            pl.BlockSpec((BLK,), lambda i: (i,)),
            pl.BlockSpec((BLK,), lambda i: (i,)),
            pl.BlockSpec((BLK, 2 * CF_DIM), lambda i: (i, 0)),
            pl.BlockSpec((BLK, 2 * CF_DIM), lambda i: (i, 0)),
            pl.BlockSpec((BLK, CD), lambda i: (i + ob, 0)),
            pl.BlockSpec((BLK, CD), lambda i: (i + ob, 0)),
            pl.BlockSpec((CD, CD), lambda i: (0, 0)),
            pl.BlockSpec((1, CD), lambda i: (0, 0)),
            pl.BlockSpec((1, CD), lambda i: (0, 0)),
            pl.BlockSpec((1, CD), lambda i: (0, 0)),
        ],
        out_specs=pl.BlockSpec((BLK,), lambda i: (i,)),
        out_shape=jax.ShapeDtypeStruct((n,), jnp.float32),
    )


def kernel(user_ids, item_ids, user_cf_weight, item_cf_weight,
           raw_user_profiles, article_content_embeddings,
           proj_W, proj_b, ln_gamma, ln_beta):
    uprof_g, icont_g = _make_sc_gather()(
        user_ids, item_ids, raw_user_profiles, article_content_embeddings)
    ut = user_cf_weight.T
    it = item_cf_weight.T
    cfcat = _tc_prep(ut, ut, it, it)
    h = BATCH // 2
    small = (proj_W, proj_b.reshape(1, CD), ln_gamma.reshape(1, CD),
             ln_beta.reshape(1, CD))
    u0, i0 = _make_sc_cfgather(h)(user_ids[:h], item_ids[:h], cfcat, uprof_g)
    u1, i1 = _make_sc_cfgather(h)(user_ids[h:], item_ids[h:], cfcat, u0)
    s0 = _make_tc_score(h, 0)(user_ids[:h], item_ids[:h], u0, i0,
                              uprof_g, icont_g, *small)
    s1 = _make_tc_score(h, h)(user_ids[h:], item_ids[h:], u1, i1,
                              uprof_g, icont_g, *small)
    return jnp.concatenate([s0, s1])


# speedup vs baseline: 1.1490x; 1.1490x over previous
import functools

import jax
import jax.numpy as jnp
from jax import lax
from jax.experimental import pallas as pl
from jax.experimental.pallas import tpu as pltpu
from jax.experimental.pallas import tpu_sc as plsc

BATCH = 16384
CF_DIM = 64
CD = 256
ALPHA = 0.5

NC = 2
NS = 16
NW = NC * NS
BPW = BATCH // NW

CF_SUB = 128
CF_NSUB = BPW // CF_SUB
PR_SUB = 64
PR_NSUB = BPW // PR_SUB

N_ROWS = 100000
N_PAD = 100096
PREP_R = 5888


def _mesh():
    return plsc.VectorSubcoreMesh(core_axis_name="c", subcore_axis_name="s",
                                  num_cores=NC, num_subcores=NS)


@functools.cache
def _make_sc_cfgather(n):
    bpw = n // NW
    nsub = bpw // CF_SUB

    @functools.partial(
        pl.kernel,
        out_type=[
            jax.ShapeDtypeStruct((n, 2 * CF_DIM), jnp.float32),
            jax.ShapeDtypeStruct((n, 2 * CF_DIM), jnp.float32),
        ],
        mesh=_mesh(),
        scratch_types=[
            pltpu.VMEM((bpw,), jnp.int32),
            pltpu.VMEM((bpw,), jnp.int32),
            pltpu.VMEM((2, CF_SUB, 2 * CF_DIM), jnp.float32),
            pltpu.VMEM((2, CF_SUB, 2 * CF_DIM), jnp.float32),
            pltpu.SemaphoreType.DMA,
            pltpu.SemaphoreType.DMA,
            pltpu.SemaphoreType.DMA,
            pltpu.SemaphoreType.DMA,
        ],
    )
    def _sc_cfgather(uids, iids, cfcat, dep, ucf_out, icf_out,
                     uid_v, iid_v, ubuf, ibuf, su0, su1, si0, si1):
        del dep
        wid = lax.axis_index("s") * NC + lax.axis_index("c")
        base = wid * bpw
        pltpu.sync_copy(uids.at[pl.ds(base, bpw)], uid_v)
        pltpu.sync_copy(iids.at[pl.ds(base, bpw)], iid_v)
        sem_u = (su0, su1)
        sem_i = (si0, si1)

        def fire(c):
            o = c * CF_SUB
            s = c % 2
            cu = pltpu.async_copy(cfcat.at[uid_v.at[pl.ds(o, CF_SUB)]],
                                  ubuf.at[s], sem_u[s])
            ci = pltpu.async_copy(cfcat.at[iid_v.at[pl.ds(o, CF_SUB)]],
                                  ibuf.at[s], sem_i[s])
            return cu, ci

        pend = fire(0)
        for c in range(nsub):
            cu, ci = pend
            if c + 1 < nsub:
                nxt = fire(c + 1)
            cu.wait()
            ci.wait()
            o = c * CF_SUB
            s = c % 2
            pltpu.sync_copy(ubuf.at[s], ucf_out.at[pl.ds(base + o, CF_SUB)])
            pltpu.sync_copy(ibuf.at[s], icf_out.at[pl.ds(base + o, CF_SUB)])
            if c + 1 < nsub:
                pend = nxt

    return _sc_cfgather


@functools.cache
def _make_sc_gather():
    @functools.partial(
        pl.kernel,
        out_type=[
            jax.ShapeDtypeStruct((BATCH, CD), jnp.float32),
            jax.ShapeDtypeStruct((BATCH, CD), jnp.float32),
        ],
        mesh=_mesh(),
        scratch_types=[
            pltpu.VMEM((BPW,), jnp.int32),
            pltpu.VMEM((BPW,), jnp.int32),
            pltpu.VMEM((2, PR_SUB, CD), jnp.float32),
            pltpu.VMEM((2, PR_SUB, CD), jnp.float32),
            pltpu.SemaphoreType.DMA,
            pltpu.SemaphoreType.DMA,
            pltpu.SemaphoreType.DMA,
            pltpu.SemaphoreType.DMA,
        ],
    )
    def _sc_gather(uids, iids, uprof, icont,
                   uprof_out, icont_out,
                   uid_v, iid_v, ubuf, ibuf, su0, su1, si0, si1):
        wid = lax.axis_index("s") * NC + lax.axis_index("c")
        base = wid * BPW
        pltpu.sync_copy(uids.at[pl.ds(base, BPW)], uid_v)
        pltpu.sync_copy(iids.at[pl.ds(base, BPW)], iid_v)
        sem_u = (su0, su1)
        sem_i = (si0, si1)

        def fire(c):
            o = c * PR_SUB
            s = c % 2
            cu = pltpu.async_copy(uprof.at[uid_v.at[pl.ds(o, PR_SUB)]],
                                  ubuf.at[s], sem_u[s])
            ci = pltpu.async_copy(icont.at[iid_v.at[pl.ds(o, PR_SUB)]],
                                  ibuf.at[s], sem_i[s])
            return cu, ci

        pend = fire(0)
        for c in range(PR_NSUB):
            cu, ci = pend
            if c + 1 < PR_NSUB:
                nxt = fire(c + 1)
            cu.wait()
            ci.wait()
            o = c * PR_SUB
            s = c % 2
            pltpu.sync_copy(ubuf.at[s], uprof_out.at[pl.ds(base + o, PR_SUB)])
            pltpu.sync_copy(ibuf.at[s], icont_out.at[pl.ds(base + o, PR_SUB)])
            if c + 1 < PR_NSUB:
                pend = nxt

    return _sc_gather


def _tc_prep_body(ut_ref, it_ref, out_ref):
    out_ref[...] = jnp.concatenate([ut_ref[...].T, it_ref[...].T], axis=1)


_tc_prep = pl.pallas_call(
    _tc_prep_body,
    grid=(N_PAD // PREP_R,),
    in_specs=[
        pl.BlockSpec((CF_DIM, PREP_R), lambda i: (0, i)),
        pl.BlockSpec((CF_DIM, PREP_R), lambda i: (0, i)),
    ],
    out_specs=pl.BlockSpec((PREP_R, 2 * CF_DIM), lambda i: (i, 0)),
    out_shape=jax.ShapeDtypeStruct((N_PAD, 2 * CF_DIM), jnp.float32),
)


BLK = 1024


def _tc_body(ucf_ref, icf_ref, uprof_ref, icont_ref, w_ref, b_ref, g_ref,
             beta_ref, out_ref):
    u = uprof_ref[...]
    h = jnp.dot(u, w_ref[...], preferred_element_type=jnp.float32)
    h = h + b_ref[...]
    mu = jnp.mean(h, axis=1, keepdims=True)
    var = jnp.mean((h - mu) * (h - mu), axis=1, keepdims=True)
    hn = (h - mu) * lax.rsqrt(var + 1e-5) * g_ref[...] + beta_ref[...]
    hg = hn * 0.5 * (1.0 + lax.erf(hn * 0.7071067811865476))
    content = jnp.sum(hg * icont_ref[...], axis=1)
    cf = jnp.sum(ucf_ref[:, :CF_DIM] * icf_ref[:, CF_DIM:], axis=1)
    out_ref[...] = ALPHA * cf + (1.0 - ALPHA) * content


@functools.cache
def _make_tc_score(n, row_off):
    ob = row_off // BLK
    return pl.pallas_call(
        _tc_body,
        grid=(n // BLK,),
        in_specs=[
            pl.BlockSpec((BLK, 2 * CF_DIM), lambda i: (i, 0)),
            pl.BlockSpec((BLK, 2 * CF_DIM), lambda i: (i, 0)),
            pl.BlockSpec((BLK, CD), lambda i: (i + ob, 0)),
            pl.BlockSpec((BLK, CD), lambda i: (i + ob, 0)),
            pl.BlockSpec((CD, CD), lambda i: (0, 0)),
            pl.BlockSpec((1, CD), lambda i: (0, 0)),
            pl.BlockSpec((1, CD), lambda i: (0, 0)),
            pl.BlockSpec((1, CD), lambda i: (0, 0)),
        ],
        out_specs=pl.BlockSpec((BLK,), lambda i: (i,)),
        out_shape=jax.ShapeDtypeStruct((n,), jnp.float32),
    )


def kernel(user_ids, item_ids, user_cf_weight, item_cf_weight,
           raw_user_profiles, article_content_embeddings,
           proj_W, proj_b, ln_gamma, ln_beta):
    uprof_g, icont_g = _make_sc_gather()(
        user_ids, item_ids, raw_user_profiles, article_content_embeddings)
    cfcat = _tc_prep(user_cf_weight.T, item_cf_weight.T)
    small = (proj_W, proj_b.reshape(1, CD), ln_gamma.reshape(1, CD),
             ln_beta.reshape(1, CD))
    ucf_g, icf_g = _make_sc_cfgather(BATCH)(user_ids, item_ids, cfcat, uprof_g)
    return _make_tc_score(BATCH, 0)(ucf_g, icf_g, uprof_g, icont_g, *small)


# speedup vs baseline: 1.1785x; 1.0257x over previous
import functools

import jax
import jax.numpy as jnp
from jax import lax
from jax.experimental import pallas as pl
from jax.experimental.pallas import tpu as pltpu
from jax.experimental.pallas import tpu_sc as plsc

BATCH = 16384
CF_DIM = 64
CD = 256
ALPHA = 0.5

NC = 2
NS = 16
NW = NC * NS
BPW = BATCH // NW

CF_SUB = 128
CF_NSUB = BPW // CF_SUB
PR_SUB = 64
PR_NSUB = BPW // PR_SUB

N_ROWS = 100000
N_PAD = 100096
PREP_R = 5888


def _mesh():
    return plsc.VectorSubcoreMesh(core_axis_name="c", subcore_axis_name="s",
                                  num_cores=NC, num_subcores=NS)


@functools.cache
def _make_sc_cfgather(n):
    bpw = n // NW
    nsub = bpw // CF_SUB

    @functools.partial(
        pl.kernel,
        out_type=[
            jax.ShapeDtypeStruct((n, 2 * CF_DIM), jnp.float32),
            jax.ShapeDtypeStruct((n, 2 * CF_DIM), jnp.float32),
        ],
        mesh=_mesh(),
        scratch_types=[
            pltpu.VMEM((bpw,), jnp.int32),
            pltpu.VMEM((bpw,), jnp.int32),
            pltpu.VMEM((2, CF_SUB, 2 * CF_DIM), jnp.float32),
            pltpu.VMEM((2, CF_SUB, 2 * CF_DIM), jnp.float32),
            pltpu.SemaphoreType.DMA,
            pltpu.SemaphoreType.DMA,
            pltpu.SemaphoreType.DMA,
            pltpu.SemaphoreType.DMA,
        ],
    )
    def _sc_cfgather(uids, iids, cfcat, dep, ucf_out, icf_out,
                     uid_v, iid_v, ubuf, ibuf, su0, su1, si0, si1):
        del dep
        wid = lax.axis_index("s") * NC + lax.axis_index("c")
        base = wid * bpw
        pltpu.sync_copy(uids.at[pl.ds(base, bpw)], uid_v)
        pltpu.sync_copy(iids.at[pl.ds(base, bpw)], iid_v)
        sem_u = (su0, su1)
        sem_i = (si0, si1)

        def fire(c):
            o = c * CF_SUB
            s = c % 2
            cu = pltpu.async_copy(cfcat.at[uid_v.at[pl.ds(o, CF_SUB)]],
                                  ubuf.at[s], sem_u[s])
            ci = pltpu.async_copy(cfcat.at[iid_v.at[pl.ds(o, CF_SUB)]],
                                  ibuf.at[s], sem_i[s])
            return cu, ci

        pend = fire(0)
        for c in range(nsub):
            cu, ci = pend
            if c + 1 < nsub:
                nxt = fire(c + 1)
            cu.wait()
            ci.wait()
            o = c * CF_SUB
            s = c % 2
            pltpu.sync_copy(ubuf.at[s], ucf_out.at[pl.ds(base + o, CF_SUB)])
            pltpu.sync_copy(ibuf.at[s], icf_out.at[pl.ds(base + o, CF_SUB)])
            if c + 1 < nsub:
                pend = nxt

    return _sc_cfgather


@functools.cache
def _make_sc_gather():
    @functools.partial(
        pl.kernel,
        out_type=[
            jax.ShapeDtypeStruct((BATCH, CD), jnp.float32),
            jax.ShapeDtypeStruct((BATCH, CD), jnp.float32),
        ],
        mesh=_mesh(),
        scratch_types=[
            pltpu.VMEM((BPW,), jnp.int32),
            pltpu.VMEM((BPW,), jnp.int32),
            pltpu.VMEM((2, PR_SUB, CD), jnp.float32),
            pltpu.VMEM((2, PR_SUB, CD), jnp.float32),
            pltpu.SemaphoreType.DMA,
            pltpu.SemaphoreType.DMA,
            pltpu.SemaphoreType.DMA,
            pltpu.SemaphoreType.DMA,
        ],
    )
    def _sc_gather(uids, iids, uprof, icont,
                   uprof_out, icont_out,
                   uid_v, iid_v, ubuf, ibuf, su0, su1, si0, si1):
        wid = lax.axis_index("s") * NC + lax.axis_index("c")
        base = wid * BPW
        pltpu.sync_copy(uids.at[pl.ds(base, BPW)], uid_v)
        pltpu.sync_copy(iids.at[pl.ds(base, BPW)], iid_v)
        sem_u = (su0, su1)
        sem_i = (si0, si1)

        def fire(c):
            o = c * PR_SUB
            s = c % 2
            cu = pltpu.async_copy(uprof.at[uid_v.at[pl.ds(o, PR_SUB)]],
                                  ubuf.at[s], sem_u[s])
            ci = pltpu.async_copy(icont.at[iid_v.at[pl.ds(o, PR_SUB)]],
                                  ibuf.at[s], sem_i[s])
            return cu, ci

        pend = fire(0)
        for c in range(PR_NSUB):
            cu, ci = pend
            if c + 1 < PR_NSUB:
                nxt = fire(c + 1)
            cu.wait()
            ci.wait()
            o = c * PR_SUB
            s = c % 2
            pltpu.sync_copy(ubuf.at[s], uprof_out.at[pl.ds(base + o, PR_SUB)])
            pltpu.sync_copy(ibuf.at[s], icont_out.at[pl.ds(base + o, PR_SUB)])
            if c + 1 < PR_NSUB:
                pend = nxt

    return _sc_gather


def _tc_prep_body(ut_ref, it_ref, out_ref):
    out_ref[...] = jnp.concatenate([ut_ref[...].T, it_ref[...].T], axis=1)


_tc_prep = pl.pallas_call(
    _tc_prep_body,
    grid=(N_PAD // PREP_R,),
    in_specs=[
        pl.BlockSpec((CF_DIM, PREP_R), lambda i: (0, i)),
        pl.BlockSpec((CF_DIM, PREP_R), lambda i: (0, i)),
    ],
    out_specs=pl.BlockSpec((PREP_R, 2 * CF_DIM), lambda i: (i, 0)),
    out_shape=jax.ShapeDtypeStruct((N_PAD, 2 * CF_DIM), jnp.float32),
)


BLK = 2048


def _tc_body(ucf_ref, icf_ref, uprof_ref, icont_ref, w_ref, b_ref, g_ref,
             beta_ref, out_ref):
    u = uprof_ref[...]
    h = jnp.dot(u, w_ref[...], preferred_element_type=jnp.float32)
    h = h + b_ref[...]
    mu = jnp.mean(h, axis=1, keepdims=True)
    var = jnp.mean((h - mu) * (h - mu), axis=1, keepdims=True)
    hn = (h - mu) * lax.rsqrt(var + 1e-5) * g_ref[...] + beta_ref[...]
    hg = hn * 0.5 * (1.0 + lax.erf(hn * 0.7071067811865476))
    content = jnp.sum(hg * icont_ref[...], axis=1)
    cf = jnp.sum(ucf_ref[:, :CF_DIM] * icf_ref[:, CF_DIM:], axis=1)
    out_ref[...] = ALPHA * cf + (1.0 - ALPHA) * content


@functools.cache
def _make_tc_score(n, row_off):
    ob = row_off // BLK
    return pl.pallas_call(
        _tc_body,
        grid=(n // BLK,),
        in_specs=[
            pl.BlockSpec((BLK, 2 * CF_DIM), lambda i: (i, 0)),
            pl.BlockSpec((BLK, 2 * CF_DIM), lambda i: (i, 0)),
            pl.BlockSpec((BLK, CD), lambda i: (i + ob, 0)),
            pl.BlockSpec((BLK, CD), lambda i: (i + ob, 0)),
            pl.BlockSpec((CD, CD), lambda i: (0, 0)),
            pl.BlockSpec((1, CD), lambda i: (0, 0)),
            pl.BlockSpec((1, CD), lambda i: (0, 0)),
            pl.BlockSpec((1, CD), lambda i: (0, 0)),
        ],
        out_specs=pl.BlockSpec((BLK,), lambda i: (i,)),
        out_shape=jax.ShapeDtypeStruct((n,), jnp.float32),
    )


def kernel(user_ids, item_ids, user_cf_weight, item_cf_weight,
           raw_user_profiles, article_content_embeddings,
           proj_W, proj_b, ln_gamma, ln_beta):
    uprof_g, icont_g = _make_sc_gather()(
        user_ids, item_ids, raw_user_profiles, article_content_embeddings)
    cfcat = _tc_prep(user_cf_weight.T, item_cf_weight.T)
    small = (proj_W, proj_b.reshape(1, CD), ln_gamma.reshape(1, CD),
             ln_beta.reshape(1, CD))
    ucf_g, icf_g = _make_sc_cfgather(BATCH)(user_ids, item_ids, cfcat, uprof_g)
    return _make_tc_score(BATCH, 0)(ucf_g, icf_g, uprof_g, icont_g, *small)
